# baseline (speedup 1.0000x reference)
import functools

import jax
import jax.numpy as jnp
from jax import lax
from jax.experimental import pallas as pl
from jax.experimental.pallas import tpu as pltpu
from jax.experimental.pallas import tpu_sc as plsc

NC, NS = 2, 16
NW = NC * NS
LANES = 16
NBUF = 4


def _sc_embed_bag(T, B, V, D):
    PD = 2 * D
    P1 = B // NW
    N2 = T - B
    P2 = N2 // NW
    CH = 112
    NCH = P2 // CH
    NGRP = NCH // NBUF
    assert B % NW == 0 and N2 % NW == 0 and P2 % CH == 0 and NCH % NBUF == 0
    assert D % LANES == 0 and P1 % LANES == 0 and P2 % LANES == 0

    mesh = plsc.VectorSubcoreMesh(core_axis_name="c", subcore_axis_name="s")

    @functools.partial(
        pl.kernel,
        out_type=(
            jax.ShapeDtypeStruct((B, PD), jnp.float32),
            jax.ShapeDtypeStruct((NW, PD), jnp.float32),
        ),
        mesh=mesh,
        compiler_params=pltpu.CompilerParams(use_tc_tiling_on_sc=True),
        scratch_types=[
            pltpu.VMEM((P1,), jnp.int32),
            pltpu.VMEM((P2,), jnp.int32),
            pltpu.VMEM((P2 + LANES,), jnp.int32),
            pltpu.VMEM((P1, PD), jnp.float32),
            pltpu.VMEM((CH, PD), jnp.float32),
            pltpu.VMEM((CH, PD), jnp.float32),
            pltpu.VMEM((CH, PD), jnp.float32),
            pltpu.VMEM((CH, PD), jnp.float32),
            pltpu.VMEM((PD,), jnp.float32),
            pltpu.SemaphoreType.DMA,
            pltpu.SemaphoreType.DMA,
            pltpu.SemaphoreType.DMA,
            pltpu.SemaphoreType.DMA,
            pltpu.SemaphoreType.DMA,
        ],
    )
    def sc_embed(text_hbm, tab2_hbm, out1_hbm, out2_hbm,
                 idx1_v, idxp2_v, par2_v, rows1_v, b0_v, b1_v, b2_v, b3_v,
                 acc_v, sem1, s0, s1, s2, s3):
        w = lax.axis_index("s") * NC + lax.axis_index("c")
        bufs = (b0_v, b1_v, b2_v, b3_v)
        sems = (s0, s1, s2, s3)

        pltpu.sync_copy(text_hbm.at[pl.ds(w * P1, P1)], idx1_v)
        for i in range(P1 // LANES):
            sl = pl.ds(i * LANES, LANES)
            idx1_v[sl] = idx1_v[sl] >> 1

        pltpu.sync_copy(text_hbm.at[pl.ds(B + w * P2, P2)], idxp2_v)

        def prep_body(i, _):
            sl = pl.ds(i * LANES, LANES)
            t = idxp2_v[sl]
            par2_v[sl] = t & 1
            idxp2_v[sl] = t >> 1
            return 0

        lax.fori_loop(0, P2 // LANES, prep_body, 0)

        cp1 = pltpu.make_async_copy(tab2_hbm.at[idx1_v], rows1_v, sem1)
        cp1.start()

        for b in range(NBUF):
            pltpu.make_async_copy(
                tab2_hbm.at[idxp2_v.at[pl.ds(b * CH, CH)]], bufs[b], sems[b]).start()

        cp1.wait()
        pltpu.sync_copy(rows1_v, out1_hbm.at[pl.ds(w * P1, P1)])

        zeros = jnp.zeros((LANES,), jnp.float32)
        acc0 = (zeros,) * (D // LANES)

        def reduce_buf(buf, base, acc):
            def row_body(r, a):
                pv = par2_v[pl.ds(base + r, LANES)]
                m = pv[0].astype(jnp.float32)
                mf = jnp.zeros((LANES,), jnp.float32) + m
                omf = 1.0 - mf
                return tuple(
                    a[k]
                    + omf * buf[r, pl.ds(k * LANES, LANES)]
                    + mf * buf[r, pl.ds(D + k * LANES, LANES)]
                    for k in range(D // LANES)
                )
            return lax.fori_loop(0, CH, row_body, acc)

        def group_body(g, acc):
            for b in range(NBUF):
                pltpu.make_async_copy(
                    tab2_hbm.at[idxp2_v.at[pl.ds(0, CH)]], bufs[b], sems[b]).wait()
                acc = reduce_buf(bufs[b], (g * NBUF + b) * CH, acc)
                pltpu.make_async_copy(
                    tab2_hbm.at[idxp2_v.at[pl.ds(((g + 1) * NBUF + b) * CH, CH)]],
                    bufs[b], sems[b]).start()
            return acc

        acc = lax.fori_loop(0, NGRP - 1, group_body, acc0)

        for b in range(NBUF):
            pltpu.make_async_copy(
                tab2_hbm.at[idxp2_v.at[pl.ds(0, CH)]], bufs[b], sems[b]).wait()
            acc = reduce_buf(bufs[b], ((NGRP - 1) * NBUF + b) * CH, acc)

        for k in range(D // LANES):
            acc_v[pl.ds(k * LANES, LANES)] = acc[k]
        for k in range(D // LANES):
            acc_v[pl.ds(D + k * LANES, LANES)] = zeros
        pltpu.sync_copy(acc_v, out2_hbm.at[w])

    return sc_embed


def _mlp_body(nbig, x2_ref, part_ref, txt_ref, w1_ref, b1_ref, w2_ref, b2_ref,
              o_ref):
    D = w1_ref.shape[1]
    x2 = x2_ref[...]
    B = x2.shape[0]
    par = (txt_ref[...] & 1).reshape(B, 1)
    parb = jnp.broadcast_to(par, (B, D))
    x = jnp.where(parb == 1, x2[:, D:], x2[:, :D])
    psum = jnp.sum(part_ref[...][:, :D], axis=0, keepdims=True)
    bigrow = (x[B - 1:B, :] + psum) * (1.0 / nbig)
    row_ids = lax.broadcasted_iota(jnp.int32, (B, 1), 0)
    x = jnp.where(row_ids == B - 1, bigrow, x)
    h = lax.dot_general(x, w1_ref[...], (((1,), (1,)), ((), ())),
                        preferred_element_type=jnp.float32)
    h = jnp.maximum(h + b1_ref[...], 0.0)
    o_ref[...] = lax.dot_general(h, w2_ref[...], (((1,), (1,)), ((), ())),
                                 preferred_element_type=jnp.float32) + b2_ref[...]


def kernel(text, offsets, table, W1, b1, W2, b2):
    T = text.shape[0]
    B = offsets.shape[0]
    V, D = table.shape
    C = W2.shape[0]

    sc_embed = _sc_embed_bag(T, B, V, D)
    tab2 = table.reshape(V // 2, 2 * D)
    out1p, part = sc_embed(text, tab2)

    nbig = float(T - B + 1)
    out = pl.pallas_call(
        functools.partial(_mlp_body, nbig),
        out_shape=jax.ShapeDtypeStruct((B, C), jnp.float32),
    )(out1p, part, text[:B], W1, b1.reshape(1, D), W2, b2.reshape(1, C))
    return out


# speedup vs baseline: 1.2626x; 1.2626x over previous
import functools

import jax
import jax.numpy as jnp
from jax import lax
from jax.experimental import pallas as pl
from jax.experimental.pallas import tpu as pltpu
from jax.experimental.pallas import tpu_sc as plsc

NC, NS = 2, 16
NW = NC * NS
LANES = 16
NBUF = 4
VBLK = 2048


def _pack_body(V, tabT_ref, w1_ref, o_ref):
    g = pl.program_id(0)
    x = tabT_ref[...]
    cols = g * VBLK + lax.broadcasted_iota(jnp.int32, x.shape, 1)
    x = jnp.where(cols < V, x, 0.0)
    z = lax.dot_general(x, w1_ref[...], (((0,), (1,)), ((), ())),
                        preferred_element_type=jnp.float32)
    h = VBLK // 2
    o_ref[:, : x.shape[0]] = z[:h, :]
    o_ref[:, x.shape[0]:] = z[h:, :]


def _sc_embed_bag(T, B, PR, D):
    PD = 2 * D
    P1 = B // NW
    N2 = T - B
    P2 = N2 // NW
    CH = 112
    NCH = P2 // CH
    NGRP = NCH // NBUF
    assert B % NW == 0 and N2 % NW == 0 and P2 % CH == 0 and NCH % NBUF == 0
    assert D % LANES == 0 and P1 % LANES == 0 and P2 % LANES == 0

    mesh = plsc.VectorSubcoreMesh(core_axis_name="c", subcore_axis_name="s")

    @functools.partial(
        pl.kernel,
        out_type=(
            jax.ShapeDtypeStruct((B, PD), jnp.float32),
            jax.ShapeDtypeStruct((NW, PD), jnp.float32),
        ),
        mesh=mesh,
        compiler_params=pltpu.CompilerParams(use_tc_tiling_on_sc=True),
        scratch_types=[
            pltpu.VMEM((P1,), jnp.int32),
            pltpu.VMEM((P2,), jnp.int32),
            pltpu.VMEM((P2 + LANES,), jnp.int32),
            pltpu.VMEM((P1, PD), jnp.float32),
            pltpu.VMEM((CH, PD), jnp.float32),
            pltpu.VMEM((CH, PD), jnp.float32),
            pltpu.VMEM((CH, PD), jnp.float32),
            pltpu.VMEM((CH, PD), jnp.float32),
            pltpu.VMEM((PD,), jnp.float32),
            pltpu.SemaphoreType.DMA,
            pltpu.SemaphoreType.DMA,
            pltpu.SemaphoreType.DMA,
            pltpu.SemaphoreType.DMA,
            pltpu.SemaphoreType.DMA,
        ],
    )
    def sc_embed(text_hbm, packed_hbm, out1_hbm, out2_hbm,
                 idx1_v, idxp2_v, par2_v, rows1_v, b0_v, b1_v, b2_v, b3_v,
                 acc_v, sem1, s0, s1, s2, s3):
        w = lax.axis_index("s") * NC + lax.axis_index("c")
        bufs = (b0_v, b1_v, b2_v, b3_v)
        sems = (s0, s1, s2, s3)

        pltpu.sync_copy(text_hbm.at[pl.ds(w * P1, P1)], idx1_v)
        for i in range(P1 // LANES):
            sl = pl.ds(i * LANES, LANES)
            t = idx1_v[sl]
            idx1_v[sl] = ((t >> 11) << 10) + (t & 1023)

        pltpu.sync_copy(text_hbm.at[pl.ds(B + w * P2, P2)], idxp2_v)

        def prep_body(i, _):
            sl = pl.ds(i * LANES, LANES)
            t = idxp2_v[sl]
            par2_v[sl] = (t >> 10) & 1
            idxp2_v[sl] = ((t >> 11) << 10) + (t & 1023)
            return 0

        lax.fori_loop(0, P2 // LANES, prep_body, 0)

        cp1 = pltpu.make_async_copy(packed_hbm.at[idx1_v], rows1_v, sem1)
        cp1.start()

        for b in range(NBUF):
            pltpu.make_async_copy(
                packed_hbm.at[idxp2_v.at[pl.ds(b * CH, CH)]],
                bufs[b], sems[b]).start()

        cp1.wait()
        pltpu.sync_copy(rows1_v, out1_hbm.at[pl.ds(w * P1, P1)])

        zeros = jnp.zeros((LANES,), jnp.float32)
        acc0 = (zeros,) * (D // LANES)

        def reduce_buf(buf, base, acc):
            def row_body(r, a):
                pv = par2_v[pl.ds(base + r, LANES)]
                m = pv[0].astype(jnp.float32)
                mf = jnp.zeros((LANES,), jnp.float32) + m
                omf = 1.0 - mf
                return tuple(
                    a[k]
                    + omf * buf[r, pl.ds(k * LANES, LANES)]
                    + mf * buf[r, pl.ds(D + k * LANES, LANES)]
                    for k in range(D // LANES)
                )
            return lax.fori_loop(0, CH, row_body, acc)

        def group_body(g, acc):
            for b in range(NBUF):
                pltpu.make_async_copy(
                    packed_hbm.at[idxp2_v.at[pl.ds(0, CH)]],
                    bufs[b], sems[b]).wait()
                acc = reduce_buf(bufs[b], (g * NBUF + b) * CH, acc)
                pltpu.make_async_copy(
                    packed_hbm.at[idxp2_v.at[pl.ds(((g + 1) * NBUF + b) * CH, CH)]],
                    bufs[b], sems[b]).start()
            return acc

        acc = lax.fori_loop(0, NGRP - 1, group_body, acc0)

        for b in range(NBUF):
            pltpu.make_async_copy(
                packed_hbm.at[idxp2_v.at[pl.ds(0, CH)]], bufs[b], sems[b]).wait()
            acc = reduce_buf(bufs[b], ((NGRP - 1) * NBUF + b) * CH, acc)

        for k in range(D // LANES):
            acc_v[pl.ds(k * LANES, LANES)] = acc[k]
        for k in range(D // LANES):
            acc_v[pl.ds(D + k * LANES, LANES)] = zeros
        pltpu.sync_copy(acc_v, out2_hbm.at[w])

    return sc_embed


def _mlp_body(nbig, x2_ref, part_ref, txt_ref, b1_ref, w2_ref, b2_ref, o_ref):
    D = b1_ref.shape[1]
    x2 = x2_ref[...]
    B = x2.shape[0]
    par = ((txt_ref[...] >> 10) & 1).reshape(B, 1)
    parb = jnp.broadcast_to(par, (B, D))
    x = jnp.where(parb == 1, x2[:, D:], x2[:, :D])
    psum = jnp.sum(part_ref[...][:, :D], axis=0, keepdims=True)
    bigrow = (x[B - 1:B, :] + psum) * (1.0 / nbig)
    row_ids = lax.broadcasted_iota(jnp.int32, (B, 1), 0)
    x = jnp.where(row_ids == B - 1, bigrow, x)
    h = jnp.maximum(x + b1_ref[...], 0.0)
    o_ref[...] = lax.dot_general(h, w2_ref[...], (((1,), (1,)), ((), ())),
                                 preferred_element_type=jnp.float32) + b2_ref[...]


def kernel(text, offsets, table, W1, b1, W2, b2):
    T = text.shape[0]
    B = offsets.shape[0]
    V, D = table.shape
    C = W2.shape[0]

    G = (V + VBLK - 1) // VBLK
    PR = G * (VBLK // 2)
    packed = pl.pallas_call(
        functools.partial(_pack_body, V),
        grid=(G,),
        in_specs=[
            pl.BlockSpec((D, VBLK), lambda g: (0, g)),
            pl.BlockSpec((D, D), lambda g: (0, 0)),
        ],
        out_specs=pl.BlockSpec((VBLK // 2, 2 * D), lambda g: (g, 0)),
        out_shape=jax.ShapeDtypeStruct((PR, 2 * D), jnp.float32),
    )(table.T, W1)

    sc_embed = _sc_embed_bag(T, B, PR, D)
    out1p, part = sc_embed(text, packed)

    nbig = float(T - B + 1)
    out = pl.pallas_call(
        functools.partial(_mlp_body, nbig),
        out_shape=jax.ShapeDtypeStruct((B, C), jnp.float32),
    )(out1p, part, text[:B], b1.reshape(1, D), W2, b2.reshape(1, C))
    return out


# speedup vs baseline: 1.2629x; 1.0003x over previous
import functools

import jax
import jax.numpy as jnp
from jax import lax
from jax.experimental import pallas as pl
from jax.experimental.pallas import tpu as pltpu
from jax.experimental.pallas import tpu_sc as plsc

NC, NS = 2, 16
NW = NC * NS
LANES = 16
NBUF = 4
VBLK = 2048


def _pack_body(V, tabT_ref, w1_ref, o_ref):
    g = pl.program_id(0)
    x = tabT_ref[...]
    cols = g * VBLK + lax.broadcasted_iota(jnp.int32, x.shape, 1)
    x = jnp.where(cols < V, x, 0.0)
    z = lax.dot_general(x, w1_ref[...], (((0,), (1,)), ((), ())),
                        preferred_element_type=jnp.float32)
    h = VBLK // 2
    o_ref[:, : x.shape[0]] = z[:h, :]
    o_ref[:, x.shape[0]:] = z[h:, :]


def _sc_embed_bag(T, B, PR, D):
    PD = 2 * D
    P1 = B // NW
    N2 = T - B
    P2 = N2 // NW
    CH = 112
    NCH = P2 // CH
    NGRP = NCH // NBUF
    assert B % NW == 0 and N2 % NW == 0 and P2 % CH == 0 and NCH % NBUF == 0
    assert D % LANES == 0 and P1 % LANES == 0 and P2 % LANES == 0

    mesh = plsc.VectorSubcoreMesh(core_axis_name="c", subcore_axis_name="s")

    @functools.partial(
        pl.kernel,
        out_type=(
            jax.ShapeDtypeStruct((B, PD), jnp.float32),
            jax.ShapeDtypeStruct((NW, PD), jnp.float32),
        ),
        mesh=mesh,
        compiler_params=pltpu.CompilerParams(use_tc_tiling_on_sc=True),
        scratch_types=[
            pltpu.VMEM((P1,), jnp.int32),
            pltpu.VMEM((P2,), jnp.int32),
            pltpu.VMEM((P2 + LANES,), jnp.int32),
            pltpu.VMEM((P1, PD), jnp.float32),
            pltpu.VMEM((CH, PD), jnp.float32),
            pltpu.VMEM((CH, PD), jnp.float32),
            pltpu.VMEM((CH, PD), jnp.float32),
            pltpu.VMEM((CH, PD), jnp.float32),
            pltpu.VMEM((PD,), jnp.float32),
            pltpu.SemaphoreType.DMA,
            pltpu.SemaphoreType.DMA,
            pltpu.SemaphoreType.DMA,
            pltpu.SemaphoreType.DMA,
            pltpu.SemaphoreType.DMA,
        ],
    )
    def sc_embed(text_hbm, packed_hbm, out1_hbm, out2_hbm,
                 idx1_v, idxp2_v, par2_v, rows1_v, b0_v, b1_v, b2_v, b3_v,
                 acc_v, sem1, s0, s1, s2, s3):
        w = lax.axis_index("s") * NC + lax.axis_index("c")
        bufs = (b0_v, b1_v, b2_v, b3_v)
        sems = (s0, s1, s2, s3)

        pltpu.sync_copy(text_hbm.at[pl.ds(w * P1, P1)], idx1_v)
        for i in range(P1 // LANES):
            sl = pl.ds(i * LANES, LANES)
            t = idx1_v[sl]
            idx1_v[sl] = ((t >> 11) << 10) + (t & 1023)

        pltpu.sync_copy(text_hbm.at[pl.ds(B + w * P2, P2)], idxp2_v)

        def prep_body(i, _):
            sl = pl.ds(i * LANES, LANES)
            t = idxp2_v[sl]
            par2_v[sl] = (t >> 10) & 1
            idxp2_v[sl] = ((t >> 11) << 10) + (t & 1023)
            return 0

        lax.fori_loop(0, P2 // LANES, prep_body, 0)

        cp1 = pltpu.make_async_copy(packed_hbm.at[idx1_v], rows1_v, sem1)
        cp1.start()

        for b in range(NBUF):
            pltpu.make_async_copy(
                packed_hbm.at[idxp2_v.at[pl.ds(b * CH, CH)]],
                bufs[b], sems[b]).start()

        cp1.wait()
        pltpu.sync_copy(rows1_v, out1_hbm.at[pl.ds(w * P1, P1)])

        zeros = jnp.zeros((LANES,), jnp.float32)
        acc0 = (zeros,) * (D // LANES)

        def reduce_buf(buf, base, acc):
            def row_body(r, a):
                pv = par2_v[pl.ds(base + r, LANES)]
                m = pv[0].astype(jnp.float32)
                mf = jnp.zeros((LANES,), jnp.float32) + m
                omf = 1.0 - mf
                return tuple(
                    a[k]
                    + omf * buf[r, pl.ds(k * LANES, LANES)]
                    + mf * buf[r, pl.ds(D + k * LANES, LANES)]
                    for k in range(D // LANES)
                )
            return lax.fori_loop(0, CH, row_body, acc)

        def group_body(g, acc):
            for b in range(NBUF):
                pltpu.make_async_copy(
                    packed_hbm.at[idxp2_v.at[pl.ds(0, CH)]],
                    bufs[b], sems[b]).wait()
                acc = reduce_buf(bufs[b], (g * NBUF + b) * CH, acc)
                pltpu.make_async_copy(
                    packed_hbm.at[idxp2_v.at[pl.ds(((g + 1) * NBUF + b) * CH, CH)]],
                    bufs[b], sems[b]).start()
            return acc

        acc = lax.fori_loop(0, NGRP - 1, group_body, acc0)

        for b in range(NBUF):
            pltpu.make_async_copy(
                packed_hbm.at[idxp2_v.at[pl.ds(0, CH)]], bufs[b], sems[b]).wait()
            acc = reduce_buf(bufs[b], ((NGRP - 1) * NBUF + b) * CH, acc)

        for k in range(D // LANES):
            acc_v[pl.ds(k * LANES, LANES)] = acc[k]
        for k in range(D // LANES):
            acc_v[pl.ds(D + k * LANES, LANES)] = zeros
        pltpu.sync_copy(acc_v, out2_hbm.at[w])

    return sc_embed


def _mlp_body(nbig, x2_ref, part_ref, txt_ref, b1_ref, w2_ref, b2_ref, o_ref):
    D = b1_ref.shape[1]
    x2 = x2_ref[...]
    B = x2.shape[0]
    par = ((txt_ref[...] >> 10) & 1).reshape(B, 1)
    parb = jnp.broadcast_to(par, (B, D))
    x = jnp.where(parb == 1, x2[:, D:], x2[:, :D])
    psum = jnp.sum(part_ref[...][:, :D], axis=0, keepdims=True)
    bigrow = (x[B - 1:B, :] + psum) * (1.0 / nbig)
    row_ids = lax.broadcasted_iota(jnp.int32, (B, 1), 0)
    x = jnp.where(row_ids == B - 1, bigrow, x)
    h = jnp.maximum(x + b1_ref[...], 0.0)
    o_ref[...] = lax.dot_general(h, w2_ref[...], (((1,), (1,)), ((), ())),
                                 preferred_element_type=jnp.float32) + b2_ref[...]


def kernel(text, offsets, table, W1, b1, W2, b2):
    T = text.shape[0]
    B = offsets.shape[0]
    V, D = table.shape
    C = W2.shape[0]

    G = (V + VBLK - 1) // VBLK
    PR = G * (VBLK // 2)
    packed = pl.pallas_call(
        functools.partial(_pack_body, V),
        grid=(G,),
        in_specs=[
            pl.BlockSpec((D, VBLK), lambda g: (0, g)),
            pl.BlockSpec((D, D), lambda g: (0, 0)),
        ],
        out_specs=pl.BlockSpec((VBLK // 2, 2 * D), lambda g: (g, 0)),
        out_shape=jax.ShapeDtypeStruct((PR, 2 * D), jnp.float32),
        compiler_params=pltpu.CompilerParams(
            dimension_semantics=("parallel",)),
    )(table.T, W1)

    sc_embed = _sc_embed_bag(T, B, PR, D)
    out1p, part = sc_embed(text, packed)

    nbig = float(T - B + 1)
    out = pl.pallas_call(
        functools.partial(_mlp_body, nbig),
        out_shape=jax.ShapeDtypeStruct((B, C), jnp.float32),
    )(out1p, part, text[:B], b1.reshape(1, D), W2, b2.reshape(1, C))
    return out
